# Initial kernel scaffold; baseline (speedup 1.0000x reference)
import jax
import jax.numpy as jnp
from jax.experimental import pallas as pl


def kernel(x, edge_index, kernel_id, W_v, g_v, b_v, W_q, g_q, b_q, codebook, W_out, g_o, b_o):
    raise NotImplementedError("write your pallas kernel here")


# speedup vs baseline: 3.2805x; 3.2805x over previous
import jax
import jax.numpy as jnp
from jax import lax
from jax.experimental import pallas as pl
from jax.experimental.pallas import tpu as pltpu
from jax.experimental.pallas import tpu_sc as plsc

N = 10000
NPAD = 10240
E = 320000
K = 27
P = 128
C = 256
CHUNK = 128
NCHUNKS = 157
EPT = NCHUNKS * CHUNK
EPAD = 16 * EPT
NSLICE = NPAD // 16


def _tc1_body(x_ref, wv_ref, gv_ref, bv_ref, wqt_ref, vcat_ref, yq_ref):
    x = x_ref[...]
    y = jnp.dot(x, wv_ref[...], preferred_element_type=jnp.float32)
    mu = jnp.mean(y, axis=0, keepdims=True)
    var = jnp.mean(y * y, axis=0, keepdims=True) - mu * mu
    v = jnp.maximum(gv_ref[...] * (y - mu) * lax.rsqrt(var + 1e-5) + bv_ref[...], 0.0)
    for q in range(4):
        vcat_ref[q * NPAD:q * NPAD + N, :] = v[:, q * 64:(q + 1) * 64]
    yq_ref[...] = jnp.dot(x, wqt_ref[...], preferred_element_type=jnp.float32)


def _tc2_body(acc_ref, wo_ref, go_ref, bo_ref, x_ref, out_ref):
    a = jnp.concatenate([acc_ref[q, 0:N, :] for q in range(4)], axis=1)
    z = jnp.dot(a, wo_ref[...], preferred_element_type=jnp.float32)
    mu = jnp.mean(z, axis=0, keepdims=True)
    var = jnp.mean(z * z, axis=0, keepdims=True) - mu * mu
    o = jnp.maximum(go_ref[...] * (z - mu) * lax.rsqrt(var + 1e-5) + bo_ref[...], 0.0)
    out_ref[...] = jnp.maximum(o + x_ref[...], 0.0)


def _sc_body(src_h, dst_h, kid_h, yqf_h, vcat_h, cbtab_h, cbs_h, qgb_h,
             acc_h,
             sv, dv, kv, gv, msg0, msg1, vrows, obuf, zbuf,
             qloc, ch0loc, ch1loc, cbt, cbs0v, cbs1v, qgbv, part, part2, partall,
             sbuf, sbuf1,
             qacc, c0acc, c1acc, pacc, acc_s, sem):
    cid = lax.axis_index("c")
    sid = lax.axis_index("s")
    lanes = lax.iota(jnp.int32, 16)
    fzero = jnp.zeros((16,), jnp.float32)
    ebase = sid * EPT
    nbase = sid * NSLICE

    pltpu.sync_copy(cbs_h.at[pl.ds(0, 32)], cbs0v.at[pl.ds(0, 32)])
    pltpu.sync_copy(cbs_h.at[pl.ds(32, 32)], cbs1v.at[pl.ds(0, 32)])
    pltpu.sync_copy(qgb_h, qgbv.at[pl.ds(0, 16)])
    pltpu.sync_copy(cbtab_h.at[pl.ds(cid * (32 * C), 32 * C)], cbt)

    def _z1(i, _):
        sbuf[pl.ds(i * 16, 16)] = fzero
        return 0
    lax.fori_loop(0, NSLICE // 16, _z1, 0)

    def _z2(i, _):
        for c8 in range(4):
            zbuf[i, pl.ds(c8 * 16, 16)] = fzero
        return 0
    lax.fori_loop(0, CHUNK, _z2, 0)

    pltpu.sync_copy(sbuf, qacc.at[pl.ds(nbase, NSLICE)])
    pltpu.sync_copy(sbuf, c0acc.at[pl.ds(nbase, NSLICE)])
    pltpu.sync_copy(sbuf, c1acc.at[pl.ds(nbase, NSLICE)])
    for j in range(NSLICE // CHUNK):
        pltpu.sync_copy(zbuf, acc_s.at[pl.ds(nbase + j * CHUNK, CHUNK)])
    plsc.subcore_barrier()

    def chunk_a(ci, _):
        off = ebase + ci * CHUNK
        pltpu.sync_copy(src_h.at[pl.ds(off, CHUNK)], sv)
        pltpu.sync_copy(kid_h.at[pl.ds(off, CHUNK)], kv)
        pltpu.sync_copy(dst_h.at[pl.ds(off, CHUNK)], dv)
        for g in range(CHUNK // 16):
            gidx = sv[pl.ds(g * 16, 16)] * 32 + kv[pl.ds(g * 16, 16)]
            gv[pl.ds(g * 16, 16)] = gidx
        pltpu.async_copy(yqf_h.at[gv], msg0, sem).wait()
        pltpu.sync_copy(msg0, qacc.at[dv], add=True)
        return 0
    lax.fori_loop(0, NCHUNKS, chunk_a, 0)
    plsc.subcore_barrier()

    pltpu.sync_copy(qacc.at[pl.ds(nbase, NSLICE)], sbuf)

    def _red(i, carry):
        s, ss = carry
        v = sbuf[pl.ds(i * 16, 16)]
        return (s + v, ss + v * v)
    s, ss = lax.fori_loop(0, NSLICE // 16, _red, (fzero, fzero))
    part[pl.ds(0, 16)] = s
    part2[pl.ds(0, 16)] = ss
    pltpu.sync_copy(part.at[pl.ds(0, 16)], pacc.at[pl.ds(sid * 32, 16)])
    pltpu.sync_copy(part2.at[pl.ds(0, 16)], pacc.at[pl.ds(sid * 32 + 16, 16)])
    plsc.subcore_barrier()
    pltpu.sync_copy(pacc, partall)
    sv_tot = fzero
    ss_tot = fzero
    for i in range(16):
        sv_tot = sv_tot + partall[pl.ds(i * 32, 16)]
        ss_tot = ss_tot + partall[pl.ds(i * 32 + 16, 16)]
    musum = fzero
    msqsum = fzero
    for l in range(16):
        idx = jnp.full((16,), l, jnp.int32)
        musum = musum + jnp.take(sv_tot, idx)
        msqsum = msqsum + jnp.take(ss_tot, idx)
    mu = musum * (1.0 / N)
    msq = msqsum * (1.0 / N)
    var = msq - mu * mu
    t = var + 1e-5
    ti = plsc.bitcast(t, jnp.int32)
    r = plsc.bitcast(jnp.int32(0x5F3759DF) - (ti >> 1), jnp.float32)
    for _ in range(3):
        r = r * (1.5 - 0.5 * t * r * r)
    qv16 = qgbv[pl.ds(0, 16)]
    gq = jnp.take(qv16, jnp.zeros((16,), jnp.int32))
    bq = jnp.take(qv16, jnp.ones((16,), jnp.int32))

    def _qn(i, _):
        v = sbuf[pl.ds(i * 16, 16)]
        sbuf[pl.ds(i * 16, 16)] = jnp.maximum(gq * (v - mu) * r + bq, 0.0)
        return 0
    lax.fori_loop(0, NSLICE // 16, _qn, 0)
    pltpu.sync_copy(sbuf, qacc.at[pl.ds(nbase, NSLICE)])
    plsc.subcore_barrier()
    pltpu.sync_copy(qacc, qloc)

    def chunk_b(ci, _):
        off = ebase + ci * CHUNK
        pltpu.sync_copy(src_h.at[pl.ds(off, CHUNK)], sv)
        pltpu.sync_copy(kid_h.at[pl.ds(off, CHUNK)], kv)
        pltpu.sync_copy(dst_h.at[pl.ds(off, CHUNK)], dv)
        for g in range(CHUNK // 16):
            sv16 = sv[pl.ds(g * 16, 16)]
            kv16 = kv[pl.ds(g * 16, 16)]
            qv = plsc.load_gather(qloc, [sv16])
            msg0[pl.ds(g * 16, 16)] = qv * plsc.load_gather(cbs0v, [kv16])
            msg1[pl.ds(g * 16, 16)] = qv * plsc.load_gather(cbs1v, [kv16])
        pltpu.sync_copy(msg0, c0acc.at[dv], add=True)
        pltpu.sync_copy(msg1, c1acc.at[dv], add=True)
        return 0
    lax.fori_loop(0, NCHUNKS, chunk_b, 0)
    plsc.subcore_barrier()

    pltpu.sync_copy(c0acc.at[pl.ds(nbase, NSLICE)], sbuf)
    pltpu.sync_copy(c1acc.at[pl.ds(nbase, NSLICE)], sbuf1)

    def _sm(i, _):
        d = sbuf1[pl.ds(i * 16, 16)] - sbuf[pl.ds(i * 16, 16)]
        ch0 = 1.0 / (1.0 + jnp.exp(d))
        sbuf[pl.ds(i * 16, 16)] = ch0
        sbuf1[pl.ds(i * 16, 16)] = 1.0 - ch0
        return 0
    lax.fori_loop(0, NSLICE // 16, _sm, 0)
    pltpu.sync_copy(sbuf, c0acc.at[pl.ds(nbase, NSLICE)])
    pltpu.sync_copy(sbuf1, c1acc.at[pl.ds(nbase, NSLICE)])
    plsc.subcore_barrier()
    pltpu.sync_copy(c0acc, ch0loc)
    pltpu.sync_copy(c1acc, ch1loc)

    for pch in range(2):
        coff = (2 * cid + pch) * NPAD
        cbase = pch * 64

        def chunk_c(ci, _):
            off = ebase + ci * CHUNK
            pltpu.sync_copy(src_h.at[pl.ds(off, CHUNK)], sv)
            pltpu.sync_copy(kid_h.at[pl.ds(off, CHUNK)], kv)
            pltpu.sync_copy(dst_h.at[pl.ds(off, CHUNK)], dv)
            for g in range(CHUNK // 16):
                sv[pl.ds(g * 16, 16)] = sv[pl.ds(g * 16, 16)] + coff
            pltpu.async_copy(vcat_h.at[sv], vrows, sem).wait()

            def edge(e, _):
                esp = jnp.full((16,), e, jnp.int32)
                dspl = plsc.load_gather(dv, [esp])
                kspl = plsc.load_gather(kv, [esp])
                ch0 = plsc.load_gather(ch0loc, [dspl])
                ch1 = plsc.load_gather(ch1loc, [dspl])
                kbase = kspl * C + cbase + lanes
                for j in range(4):
                    cb0 = plsc.load_gather(cbt, [kbase + j * 16])
                    cb1 = plsc.load_gather(cbt, [kbase + (j * 16 + P)])
                    vr = vrows[e, pl.ds(j * 16, 16)]
                    obuf[e, pl.ds(j * 16, 16)] = vr * (ch0 * cb0 + ch1 * cb1)
                return 0
            lax.fori_loop(0, CHUNK, edge, 0)
            pltpu.sync_copy(obuf, acc_s.at[dv], add=True)
            return 0
        lax.fori_loop(0, NCHUNKS, chunk_c, 0)
        plsc.subcore_barrier()

        pltpu.sync_copy(acc_s.at[pl.ds(nbase, NSLICE)],
                        acc_h.at[2 * cid + pch, pl.ds(nbase, NSLICE)])
        if pch == 0:
            for j in range(NSLICE // CHUNK):
                pltpu.sync_copy(zbuf, acc_s.at[pl.ds(nbase + j * CHUNK, CHUNK)])
            plsc.subcore_barrier()


def kernel(x, edge_index, kernel_id, W_v, g_v, b_v, W_q, g_q, b_q, codebook, W_out, g_o, b_o):
    f32 = jnp.float32
    npd = EPAD - E
    src = jnp.concatenate([edge_index[0], jnp.zeros((npd,), jnp.int32)])
    dst = jnp.concatenate([edge_index[1], jnp.zeros((npd,), jnp.int32)])
    kid = jnp.concatenate([kernel_id.astype(jnp.int32),
                           jnp.full((npd,), K, jnp.int32)])

    wqt = jnp.zeros((P, 32), f32).at[:, :K].set(W_q.T)
    cbsum = codebook.sum(-1)
    cbs = jnp.zeros((64,), f32).at[:K].set(cbsum[0]).at[32:32 + K].set(cbsum[1])
    core_tab = jnp.transpose(codebook.reshape(2, K, 2, P), (2, 1, 0, 3)).reshape(2, K, C)
    cbtab = jnp.zeros((2, 32, C), f32).at[:, :K].set(core_tab)
    qgb = jnp.zeros((16,), f32).at[0].set(g_q[0]).at[1].set(b_q[0])

    vcat, yq = pl.pallas_call(
        _tc1_body,
        out_shape=[jax.ShapeDtypeStruct((4 * NPAD, 64), f32),
                   jax.ShapeDtypeStruct((N, 32), f32)],
    )(x, W_v, g_v[None, :], b_v[None, :], wqt)

    mesh = plsc.VectorSubcoreMesh(core_axis_name="c", subcore_axis_name="s",
                                  num_cores=2, num_subcores=16)
    acc = pl.kernel(
        _sc_body,
        out_type=jax.ShapeDtypeStruct((4, NPAD, 64), f32),
        mesh=mesh,
        compiler_params=pltpu.CompilerParams(needs_layout_passes=False,
                                             use_tc_tiling_on_sc=False),
        scratch_types=[
            pltpu.VMEM((CHUNK,), jnp.int32),
            pltpu.VMEM((CHUNK,), jnp.int32),
            pltpu.VMEM((CHUNK,), jnp.int32),
            pltpu.VMEM((CHUNK,), jnp.int32),
            pltpu.VMEM((CHUNK,), f32),
            pltpu.VMEM((CHUNK,), f32),
            pltpu.VMEM((CHUNK, 64), f32),
            pltpu.VMEM((CHUNK, 64), f32),
            pltpu.VMEM((CHUNK, 64), f32),
            pltpu.VMEM((NPAD,), f32),
            pltpu.VMEM((NPAD,), f32),
            pltpu.VMEM((NPAD,), f32),
            pltpu.VMEM((32 * C,), f32),
            pltpu.VMEM((128,), f32),
            pltpu.VMEM((128,), f32),
            pltpu.VMEM((128,), f32),
            pltpu.VMEM((128,), f32),
            pltpu.VMEM((128,), f32),
            pltpu.VMEM((512,), f32),
            pltpu.VMEM((NSLICE,), f32),
            pltpu.VMEM((NSLICE,), f32),
            pltpu.VMEM_SHARED((NPAD,), f32),
            pltpu.VMEM_SHARED((NPAD,), f32),
            pltpu.VMEM_SHARED((NPAD,), f32),
            pltpu.VMEM_SHARED((512,), f32),
            pltpu.VMEM_SHARED((NPAD, 64), f32),
            pltpu.SemaphoreType.DMA,
        ],
    )(src, dst, kid, yq.reshape(-1), vcat, cbtab.reshape(-1), cbs, qgb)

    out = pl.pallas_call(
        _tc2_body,
        out_shape=jax.ShapeDtypeStruct((N, P), f32),
    )(acc, W_out, g_o[None, :], b_o[None, :], x)
    return out
